# Initial kernel scaffold; baseline (speedup 1.0000x reference)
import jax
import jax.numpy as jnp
from jax.experimental import pallas as pl


def kernel(x, edge_index, batch, lin1_W, lin1_b, conv1_W, conv1_b, conv2_W, conv2_b, conv3_W, conv3_b, lin2_W, lin2_b, lin3_W, lin3_b, lin4_W, lin4_b, lin5_W, lin5_b, lin6_W, lin6_b):
    raise NotImplementedError("write your pallas kernel here")


# speedup vs baseline: 10.0536x; 10.0536x over previous
import functools

import jax
import jax.numpy as jnp
from jax import lax
from jax.experimental import pallas as pl
from jax.experimental.pallas import tpu as pltpu
from jax.experimental.pallas import tpu_sc as plsc

_NC = 2
_NS = 16
_NW = _NC * _NS
_CH = 128
_GK = 4
_W = 16


def _make_deg_kernel(NPAD, PT):
    RPT = NPAD // _NS
    mesh = plsc.VectorSubcoreMesh(core_axis_name="c", subcore_axis_name="s")

    @functools.partial(
        pl.kernel,
        out_type=jax.ShapeDtypeStruct((_NC, NPAD, 1), jnp.float32),
        mesh=mesh,
        compiler_params=pltpu.CompilerParams(use_tc_tiling_on_sc=False),
        scratch_types=[
            pltpu.VMEM((PT, _CH), jnp.int32),
            pltpu.VMEM((_CH, 1), jnp.float32),
            pltpu.VMEM_SHARED((NPAD, 1), jnp.float32),
            pltpu.SemaphoreType.DMA,
        ],
    )
    def deg_kernel(col_hbm, ones_hbm, zeros_hbm, out_hbm, colv, onesv, acc, ssem):
        c = lax.axis_index("c")
        s = lax.axis_index("s")
        wid = s * _NC + c
        r0 = s * RPT
        pltpu.sync_copy(col_hbm.at[wid], colv)
        pltpu.sync_copy(ones_hbm, onesv)
        pltpu.sync_copy(zeros_hbm, acc.at[pl.ds(r0, RPT)])
        plsc.subcore_barrier()

        def group(gi, carry):
            base = gi * _GK
            descs = [
                pltpu.async_copy(onesv, acc.at[colv.at[base + k]], ssem, add=True)
                for k in range(_GK)
            ]
            for d in descs:
                d.wait()
            return carry

        lax.fori_loop(0, PT // _GK, group, 0)
        plsc.subcore_barrier()
        pltpu.sync_copy(acc.at[pl.ds(r0, RPT)], out_hbm.at[c, pl.ds(r0, RPT)])

    return deg_kernel


def _make_scatter_kernel(NPAD, PT, nsl):
    RPT = NPAD // _NS
    W = _W
    mesh = plsc.VectorSubcoreMesh(core_axis_name="c", subcore_axis_name="s")

    @functools.partial(
        pl.kernel,
        out_type=[jax.ShapeDtypeStruct((_NC, NPAD, W), jnp.float32)] * nsl,
        mesh=mesh,
        compiler_params=pltpu.CompilerParams(use_tc_tiling_on_sc=False),
        scratch_types=(
            [pltpu.VMEM((PT, _CH), jnp.int32)] * 2
            + [pltpu.VMEM((_CH, W), jnp.float32)] * _GK
            + [
                pltpu.VMEM_SHARED((NPAD, W), jnp.float32),
                pltpu.SemaphoreType.DMA,
                pltpu.SemaphoreType.DMA,
            ]
        ),
    )
    def scat(*refs):
        row_hbm, col_hbm, zeros_hbm = refs[:3]
        us = refs[3:3 + nsl]
        outs = refs[3 + nsl:3 + 2 * nsl]
        rowv, colv = refs[3 + 2 * nsl:5 + 2 * nsl]
        gbufs = refs[5 + 2 * nsl:5 + 2 * nsl + _GK]
        acc, gsem, ssem = refs[5 + 2 * nsl + _GK:]
        c = lax.axis_index("c")
        s = lax.axis_index("s")
        wid = s * _NC + c
        r0 = s * RPT
        pltpu.sync_copy(row_hbm.at[wid], rowv)
        pltpu.sync_copy(col_hbm.at[wid], colv)
        for si in range(nsl):
            pltpu.sync_copy(zeros_hbm, acc.at[pl.ds(r0, RPT)])
            plsc.subcore_barrier()

            def group(gi, carry, _u=us[si]):
                base = gi * _GK
                gd = [
                    pltpu.async_copy(_u.at[rowv.at[base + k]], gbufs[k], gsem)
                    for k in range(_GK)
                ]
                for d in gd:
                    d.wait()
                sd = [
                    pltpu.async_copy(
                        gbufs[k], acc.at[colv.at[base + k]], ssem, add=True
                    )
                    for k in range(_GK)
                ]
                for d in sd:
                    d.wait()
                return carry

            lax.fori_loop(0, PT // _GK, group, 0)
            plsc.subcore_barrier()
            pltpu.sync_copy(acc.at[pl.ds(r0, RPT)], outs[si].at[c, pl.ds(r0, RPT)])
            if si + 1 < nsl:
                plsc.subcore_barrier()

    return scat


def _full(shape):
    nd = len(shape)
    return pl.BlockSpec(shape, lambda i, _n=nd: (0,) * _n)


def _relu(v):
    return jnp.maximum(v, 0.0)


def _dot(a, b):
    return jnp.dot(a, b, preferred_element_type=jnp.float32)


def _prep_call(BLK, N, NPAD, x, W1, b1, Wc1p, degp):
    nblk = N // BLK
    din, dmid = W1.shape
    dpad = Wc1p.shape[1]

    def body(x_ref, w1_ref, b1_ref, wc1_ref, degp_ref, dinv_ref, hw_ref, *u_refs):
        deg = 1.0 + degp_ref[0] + degp_ref[1]
        dinv = lax.rsqrt(deg)
        dinv_ref[...] = dinv
        h1 = _relu(_dot(x_ref[...], w1_ref[...]) + b1_ref[...])
        hw = _dot(h1, wc1_ref[...])
        hw_ref[...] = hw
        u = hw * dinv
        for k, ur in enumerate(u_refs):
            ur[...] = u[:, _W * k:_W * (k + 1)]

    nsl = dpad // _W
    return pl.pallas_call(
        body,
        grid=(nblk,),
        in_specs=[
            pl.BlockSpec((BLK, din), lambda i: (i, 0)),
            _full((din, dmid)),
            _full((1, dmid)),
            _full((dmid, dpad)),
            pl.BlockSpec((2, BLK, 1), lambda i: (0, i, 0)),
        ],
        out_specs=[
            pl.BlockSpec((BLK, 1), lambda i: (i, 0)),
            pl.BlockSpec((BLK, dpad), lambda i: (i, 0)),
        ] + [pl.BlockSpec((BLK, _W), lambda i: (i, 0))] * nsl,
        out_shape=[
            jax.ShapeDtypeStruct((N, 1), jnp.float32),
            jax.ShapeDtypeStruct((N, dpad), jnp.float32),
        ] + [jax.ShapeDtypeStruct((N, _W), jnp.float32)] * nsl,
    )(x, W1, b1, Wc1p, degp)


def _mid_call(BLK, N, NPAD, d_in, accs, hw, dinv, b, Wnp):
    nblk = N // BLK
    nin = len(accs)
    hwpad = hw.shape[1]
    dpad = Wnp.shape[1]
    nsl = dpad // _W

    def body(*refs):
        acc_refs = refs[:nin]
        hw_ref, dinv_ref, b_ref, wn_ref = refs[nin:nin + 4]
        hwn_ref = refs[nin + 4]
        u_refs = refs[nin + 5:]
        agg = jnp.concatenate([a[0] + a[1] for a in acc_refs], axis=1)[:, :d_in]
        d = dinv_ref[...]
        h = _relu(d * (agg + d * hw_ref[...][:, :d_in]) + b_ref[...])
        hwn = _dot(h, wn_ref[...])
        hwn_ref[...] = hwn
        u = hwn * d
        for k, ur in enumerate(u_refs):
            ur[...] = u[:, _W * k:_W * (k + 1)]

    return pl.pallas_call(
        body,
        grid=(nblk,),
        in_specs=[pl.BlockSpec((2, BLK, _W), lambda i: (0, i, 0))] * nin + [
            pl.BlockSpec((BLK, hwpad), lambda i: (i, 0)),
            pl.BlockSpec((BLK, 1), lambda i: (i, 0)),
            _full((1, d_in)),
            _full((d_in, dpad)),
        ],
        out_specs=[pl.BlockSpec((BLK, dpad), lambda i: (i, 0))]
        + [pl.BlockSpec((BLK, _W), lambda i: (i, 0))] * nsl,
        out_shape=[jax.ShapeDtypeStruct((N, dpad), jnp.float32)]
        + [jax.ShapeDtypeStruct((N, _W), jnp.float32)] * nsl,
    )(*accs, hw, dinv, b, Wnp)


def _final_call(BLK, N, NPAD, G, d_in, accs, hw, dinv, b, batch3, head):
    nblk = N // BLK
    nin = len(accs)
    hwpad = hw.shape[1]
    nhead = len(head)
    ncls = head[-2].shape[1]

    def body(*refs):
        acc_refs = refs[:nin]
        hw_ref, dinv_ref, b_ref, batch_ref = refs[nin:nin + 4]
        head_refs = refs[nin + 4:nin + 4 + nhead]
        out_ref = refs[nin + 4 + nhead]
        accg = refs[nin + 5 + nhead]
        i = pl.program_id(0)

        @pl.when(i == 0)
        def _init():
            accg[...] = jnp.full((G, d_in), -jnp.inf, jnp.float32)

        agg = jnp.concatenate([a[0] + a[1] for a in acc_refs], axis=1)[:, :d_in]
        d = dinv_ref[...]
        h = _relu(d * (agg + d * hw_ref[...][:, :d_in]) + b_ref[...])
        bb = batch_ref[0]
        gmin = jnp.min(bb)
        gmax = jnp.max(bb)
        rid = lax.broadcasted_iota(jnp.int32, (G, 1), 0)

        def gbody(g, carry):
            masked = jnp.where(bb == g, h, -jnp.inf)
            m = jnp.max(masked, axis=0, keepdims=True)
            accg[...] = jnp.where(rid == g, jnp.maximum(accg[...], m), accg[...])
            return carry

        lax.fori_loop(gmin, gmax + 1, gbody, 0)

        @pl.when(i == nblk - 1)
        def _head():
            g = accg[...]
            for k in range(0, nhead - 2, 2):
                g = _relu(_dot(g, head_refs[k][...]) + head_refs[k + 1][...])
            logits = _dot(g, head_refs[nhead - 2][...]) + head_refs[nhead - 1][...]
            mx = jnp.max(logits, axis=0, keepdims=True)
            e = jnp.exp(logits - mx)
            out_ref[...] = e / jnp.sum(e, axis=0, keepdims=True)

    return pl.pallas_call(
        body,
        grid=(nblk,),
        in_specs=[pl.BlockSpec((2, BLK, _W), lambda i: (0, i, 0))] * nin + [
            pl.BlockSpec((BLK, hwpad), lambda i: (i, 0)),
            pl.BlockSpec((BLK, 1), lambda i: (i, 0)),
            _full((1, d_in)),
            pl.BlockSpec((1, BLK, 1), lambda i: (i, 0, 0)),
        ] + [_full(hr.shape) for hr in head],
        out_specs=pl.BlockSpec((G, ncls), lambda i: (0, 0)),
        out_shape=jax.ShapeDtypeStruct((G, ncls), jnp.float32),
        scratch_shapes=[pltpu.VMEM((G, d_in), jnp.float32)],
    )(*accs, hw, dinv, b, batch3, *head)


def kernel(x, edge_index, batch, lin1_W, lin1_b, conv1_W, conv1_b, conv2_W,
           conv2_b, conv3_W, conv3_b, lin2_W, lin2_b, lin3_W, lin3_b, lin4_W,
           lin4_b, lin5_W, lin5_b, lin6_W, lin6_b):
    N = x.shape[0]
    E = edge_index.shape[1]
    G = 128
    BLK = 400
    PT = -(-E // (_NW * _CH))
    Ep = _NW * PT * _CH
    NPAD = -(-(N + 1) // 128) * 128
    RPT = NPAD // _NS

    f32 = jnp.float32
    row = edge_index[0]
    col = edge_index[1]
    rowp = jnp.concatenate([row, jnp.zeros((Ep - E,), jnp.int32)])
    colp = jnp.concatenate([col, jnp.full((Ep - E,), N, jnp.int32)])
    row3 = rowp.reshape(_NW, PT, _CH)
    col3 = colp.reshape(_NW, PT, _CH)
    batch3 = batch.reshape(N // BLK, BLK, 1)

    zerosW = jnp.zeros((RPT, _W), f32)
    zeros1 = jnp.zeros((RPT, 1), f32)
    ones1 = jnp.ones((_CH, 1), f32)

    Wc1p = jnp.pad(conv1_W, ((0, 0), (0, 96 - conv1_W.shape[1])))
    Wc2p = jnp.pad(conv2_W, ((0, 0), (0, 80 - conv2_W.shape[1])))
    Wc3p = jnp.pad(conv3_W, ((0, 0), (0, 64 - conv3_W.shape[1])))
    b1r = lin1_b.reshape(1, -1)
    bc1r = conv1_b.reshape(1, -1)
    bc2r = conv2_b.reshape(1, -1)
    bc3r = conv3_b.reshape(1, -1)
    head = [lin2_W, lin2_b.reshape(1, -1), lin3_W, lin3_b.reshape(1, -1),
            lin4_W, lin4_b.reshape(1, -1), lin5_W, lin5_b.reshape(1, -1),
            lin6_W, lin6_b.reshape(1, -1)]

    degp = _make_deg_kernel(NPAD, PT)(col3, ones1, zeros1)
    dinv, hw1, *u1 = _prep_call(BLK, N, NPAD, x, lin1_W, b1r, Wc1p, degp)

    scat6 = _make_scatter_kernel(NPAD, PT, 6)
    scat5 = _make_scatter_kernel(NPAD, PT, 5)
    scat4 = _make_scatter_kernel(NPAD, PT, 4)

    a1 = scat6(row3, col3, zerosW, *u1)
    hw2, *u2 = _mid_call(BLK, N, NPAD, 90, tuple(a1), hw1, dinv, bc1r, Wc2p)
    a2 = scat5(row3, col3, zerosW, *u2)
    hw3, *u3 = _mid_call(BLK, N, NPAD, 70, tuple(a2), hw2, dinv, bc2r, Wc3p)
    a3 = scat4(row3, col3, zerosW, *u3)
    out = _final_call(BLK, N, NPAD, G, 50, tuple(a3), hw3, dinv, bc3r, batch3,
                      head)
    return out
